# Initial kernel scaffold; baseline (speedup 1.0000x reference)
import jax
import jax.numpy as jnp
from jax.experimental import pallas as pl


def kernel(x, h, edge_attr, embed_w, embed_b, A_w, A_b, B_w, Ws_w, res_w, upd_w, upd_b, bn_g, bn_b, edge_index, batch):
    raise NotImplementedError("write your pallas kernel here")


# speedup vs baseline: 4.2219x; 4.2219x over previous
import functools
import jax
import jax.numpy as jnp
from jax import lax
from jax.experimental import pallas as pl
from jax.experimental.pallas import tpu as pltpu
from jax.experimental.pallas import tpu_sc as plsc

N = 10000
E = 160000
HID = 256
T = 4
RANK = 16
HALF = 128
NS = 16
SROWS = 2 * N + 96
SDUMP = 2 * N
GROWS = 10240
DROWS = 10240
F32 = jnp.float32

_MESH = plsc.VectorSubcoreMesh(core_axis_name="c", subcore_axis_name="s")


def _stats_body(colp, etp, distp, zs, dt_out,
                dt_acc, cbuf, ebuf, dbuf, pay,
                cbuf_t, ebuf_t, dbuf_t, pay_t, sem):
    c = lax.axis_index("c")
    s = lax.axis_index("s")
    pltpu.sync_copy(zs.at[pl.ds(0, 640)], dt_acc.at[pl.ds(s * 640, 640)])
    plsc.subcore_barrier()
    base = (c * NS + s) * 5000
    ones16 = jnp.ones((16,), F32)
    iota16 = lax.iota(jnp.int32, 16)

    def chunk(j, carry):
        b = base + j * 128
        pltpu.sync_copy(colp.at[pl.ds(b, 128)], cbuf)
        pltpu.sync_copy(etp.at[pl.ds(b, 128)], ebuf)
        pltpu.sync_copy(distp.at[pl.ds(b, 128)], dbuf)
        pltpu.sync_copy(zs.at[pl.ds(0, 128)], pay)
        for g in range(8):
            kk = g * 16
            tv = ebuf[pl.ds(kk, 16)]
            dv = dbuf[pl.ds(kk, 16)]
            rv = iota16 + kk
            plsc.store_scatter(pay, [rv, tv], ones16)
            plsc.store_scatter(pay, [rv, tv + 4], dv)
        pltpu.sync_copy(pay, dt_acc.at[cbuf], add=True)
        return carry

    lax.fori_loop(0, 39, chunk, 0)
    b = base + 4992
    pltpu.sync_copy(colp.at[pl.ds(b, 16)], cbuf_t)
    pltpu.sync_copy(etp.at[pl.ds(b, 16)], ebuf_t)
    pltpu.sync_copy(distp.at[pl.ds(b, 16)], dbuf_t)
    pltpu.sync_copy(zs.at[pl.ds(0, 16)], pay_t)
    cv = jnp.where(iota16 < 8, cbuf_t[...], N)
    tv = ebuf_t[...]
    dv = dbuf_t[...]
    plsc.store_scatter(pay_t, [iota16, tv], ones16)
    plsc.store_scatter(pay_t, [iota16, tv + 4], dv)
    pltpu.sync_copy(pay_t, dt_acc.at[cv], add=True)
    plsc.subcore_barrier()
    pltpu.sync_copy(dt_acc.at[pl.ds(s * 640, 640)],
                    dt_out.at[pl.ds(c * DROWS + s * 640, 640)])


_sc_stats = functools.partial(
    pl.kernel, _stats_body,
    out_type=jax.ShapeDtypeStruct((2 * DROWS, 16), F32),
    mesh=_MESH,
    compiler_params=pltpu.CompilerParams(use_tc_tiling_on_sc=False, needs_layout_passes=False),
    scratch_types=[
        pltpu.VMEM_SHARED((DROWS, 16), F32),
        pltpu.VMEM((128,), jnp.int32),
        pltpu.VMEM((128,), jnp.int32),
        pltpu.VMEM((128,), F32),
        pltpu.VMEM((128, 16), F32),
        pltpu.VMEM((16,), jnp.int32),
        pltpu.VMEM((16,), jnp.int32),
        pltpu.VMEM((16,), F32),
        pltpu.VMEM((16, 16), F32),
        pltpu.SemaphoreType.DMA,
    ],
)()


def _layer_body(ncat, p4, gg, gs, sg, ss, zg, zs, g_out, s_out,
                g_acc, s_acc, gidx, cidx, sgidx, ssidx, gbuf, pbuf,
                gidx_t, cidx_t, sgidx_t, ssidx_t, gbuf_t, pbuf_t, sem, sem2):
    c = lax.axis_index("c")
    s = lax.axis_index("s")
    pltpu.sync_copy(zg, g_acc.at[pl.ds(s * 640, 640)])
    pltpu.sync_copy(zs, s_acc.at[pl.ds(s * 1256, 1256)])
    plsc.subcore_barrier()
    base = s * 10000
    cE = c * E

    def chunk(j, carry):
        b = base + j * 128
        pltpu.sync_copy(gg.at[pl.ds(cE + b, 128)], gidx)
        pltpu.sync_copy(gs.at[pl.ds(b, 128)], cidx)
        pltpu.sync_copy(sg.at[pl.ds(b, 128)], sgidx)
        pltpu.sync_copy(ss.at[pl.ds(cE + b, 128)], ssidx)
        pltpu.async_copy(ncat.at[gidx], gbuf, sem).wait()
        pltpu.sync_copy(gbuf, g_acc.at[cidx], add=True)
        pltpu.async_copy(p4.at[sgidx], pbuf, sem2).wait()
        pltpu.sync_copy(pbuf, s_acc.at[ssidx], add=True)
        return carry

    lax.fori_loop(0, 78, chunk, 0)
    b = base + 9984
    pltpu.sync_copy(gg.at[pl.ds(cE + b, 16)], gidx_t)
    pltpu.sync_copy(gs.at[pl.ds(b, 16)], cidx_t)
    pltpu.sync_copy(sg.at[pl.ds(b, 16)], sgidx_t)
    pltpu.sync_copy(ss.at[pl.ds(cE + b, 16)], ssidx_t)
    pltpu.async_copy(ncat.at[gidx_t], gbuf_t, sem).wait()
    pltpu.sync_copy(gbuf_t, g_acc.at[cidx_t], add=True)
    pltpu.async_copy(p4.at[sgidx_t], pbuf_t, sem2).wait()
    pltpu.sync_copy(pbuf_t, s_acc.at[ssidx_t], add=True)
    plsc.subcore_barrier()
    pltpu.sync_copy(g_acc.at[pl.ds(s * 640, 640)],
                    g_out.at[pl.ds(c * GROWS + s * 640, 640)])
    pltpu.sync_copy(s_acc.at[pl.ds(s * 1256, 1256)],
                    s_out.at[pl.ds(c * SROWS + s * 1256, 1256)])


_sc_layer = functools.partial(
    pl.kernel, _layer_body,
    out_type=(jax.ShapeDtypeStruct((2 * GROWS, HALF), F32),
              jax.ShapeDtypeStruct((2 * SROWS, 16), F32)),
    mesh=_MESH,
    compiler_params=pltpu.CompilerParams(use_tc_tiling_on_sc=False, needs_layout_passes=False),
    scratch_types=[
        pltpu.VMEM_SHARED((GROWS, HALF), F32),
        pltpu.VMEM_SHARED((SROWS, 16), F32),
        pltpu.VMEM((128,), jnp.int32),
        pltpu.VMEM((128,), jnp.int32),
        pltpu.VMEM((128,), jnp.int32),
        pltpu.VMEM((128,), jnp.int32),
        pltpu.VMEM((128, HALF), F32),
        pltpu.VMEM((128, 16), F32),
        pltpu.VMEM((16,), jnp.int32),
        pltpu.VMEM((16,), jnp.int32),
        pltpu.VMEM((16,), jnp.int32),
        pltpu.VMEM((16,), jnp.int32),
        pltpu.VMEM((16, HALF), F32),
        pltpu.VMEM((16, 16), F32),
        pltpu.SemaphoreType.DMA,
        pltpu.SemaphoreType.DMA,
    ],
)()


_TN = 1000
_GRID = N // _TN


def _full(shape):
    return pl.BlockSpec(shape, lambda i: (0, 0))


def _rows(width):
    return pl.BlockSpec((_TN, width), lambda i: (i, 0))


def _tca_body(h, ewT, eb, bsT, node_o, p4_o):
    node = jnp.dot(h[...], ewT[...], preferred_element_type=F32) + eb[...]
    node_o[...] = node
    p4_o[...] = jnp.dot(node, bsT[...], preferred_element_type=F32)


def _tc_a(h, ewT, eb, bsT):
    return pl.pallas_call(
        _tca_body,
        grid=(_GRID,),
        in_specs=[_rows(HID), _full((HID, HID)), _full((1, HID)),
                  _full((HID, 64))],
        out_specs=[_rows(HID), _rows(64)],
        out_shape=[jax.ShapeDtypeStruct((N, HID), F32),
                   jax.ShapeDtypeStruct((N, 64), F32)],
    )(h, ewT, eb, bsT)


def _tcb_body(G, node, S, st0, st1, WsT, WtT, wd, BtT, bd, Acat, R1, R2, R3,
              araw_o, sums_o):
    i = pl.program_id(0)
    st = st0[...] + st1[...]
    cnt = jnp.sum(st[:, 0:4], axis=1, keepdims=True)
    dsum = jnp.sum(st[:, 4:8], axis=1, keepdims=True)
    denom = jnp.maximum(cnt, 1.0)
    nodev = node[...]
    ap = (jnp.dot(G[...], WsT[...], preferred_element_type=F32)
          + cnt * jnp.dot(nodev, WtT[...], preferred_element_type=F32)
          + dsum * wd[...])
    Q4 = jnp.dot(nodev, BtT[...], preferred_element_type=F32)
    Z = (S[...] + jnp.dot(st, R1[...], preferred_element_type=F32) * Q4
         + jnp.dot(st, R2[...], preferred_element_type=F32) * bd[...])
    lr = (jnp.dot(Z, Acat[...], preferred_element_type=F32)
          + jnp.dot(st, R3[...], preferred_element_type=F32))
    araw = (ap + lr) / denom
    araw_o[...] = araw
    s1 = jnp.sum(araw, axis=0, keepdims=True)
    s2 = jnp.sum(araw * araw, axis=0, keepdims=True)
    pad = jnp.concatenate([s1, s2, jnp.zeros((6, HID), F32)], axis=0)

    @pl.when(i == 0)
    def _():
        sums_o[...] = pad

    @pl.when(i > 0)
    def _():
        sums_o[...] = sums_o[...] + pad


def _tc_b(G, node, S, st0, st1, WsT, WtT, wd, BtT, bd, Acat, R1, R2, R3):
    return pl.pallas_call(
        _tcb_body,
        grid=(_GRID,),
        in_specs=[_rows(HID), _rows(HID), _rows(64), _rows(16), _rows(16),
                  _full((HID, HID)), _full((HID, HID)), _full((1, HID)),
                  _full((HID, 64)), _full((1, 64)), _full((64, HID)),
                  _full((16, 64)), _full((16, 64)), _full((16, HID))],
        out_specs=[_rows(HID), pl.BlockSpec((8, HID), lambda i: (0, 0))],
        out_shape=[jax.ShapeDtypeStruct((N, HID), F32),
                   jax.ShapeDtypeStruct((8, HID), F32)],
    )(G, node, S, st0, st1, WsT, WtT, wd, BtT, bd, Acat, R1, R2, R3)


def _tcc_body(araw, sums, node, bng, bnb, resT, unT, uaT, ub, bsnT,
              node_o, p4_o):
    mu = sums[0:1, :] * (1.0 / N)
    var = sums[1:2, :] * (1.0 / N) - mu * mu
    inv = lax.rsqrt(var + 1e-5)
    aggr = (araw[...] - mu) * (inv * bng[...]) + bnb[...]
    nodev = node[...]
    pre = (jnp.dot(nodev, unT[...], preferred_element_type=F32)
           + jnp.dot(aggr, uaT[...], preferred_element_type=F32) + ub[...])
    nn = (jnp.dot(nodev, resT[...], preferred_element_type=F32)
          + jnp.maximum(pre, 0.0))
    node_o[...] = nn
    p4_o[...] = jnp.dot(nn, bsnT[...], preferred_element_type=F32)


def _tc_c(araw, sums, node, bng, bnb, resT, unT, uaT, ub, bsnT):
    return pl.pallas_call(
        _tcc_body,
        grid=(_GRID,),
        in_specs=[_rows(HID), pl.BlockSpec((8, HID), lambda i: (0, 0)),
                  _rows(HID), _full((1, HID)), _full((1, HID)),
                  _full((HID, HID)), _full((HID, HID)), _full((HID, HID)),
                  _full((1, HID)), _full((HID, 64))],
        out_specs=[_rows(HID), _rows(64)],
        out_shape=[jax.ShapeDtypeStruct((N, HID), F32),
                   jax.ShapeDtypeStruct((N, 64), F32)],
    )(araw, sums, node, bng, bnb, resT, unT, uaT, ub, bsnT)


def kernel(x, h, edge_attr, embed_w, embed_b, A_w, A_b, B_w, Ws_w, res_w,
           upd_w, upd_b, bn_g, bn_b, edge_index, batch):
    row = edge_index[0].astype(jnp.int32)
    col = edge_index[1].astype(jnp.int32)
    et = edge_attr[:, 0].astype(jnp.int32)
    dist = edge_attr[:, 1].astype(F32)

    gg = jnp.concatenate([row, row + N])
    sg = row * T + et
    ss0 = jnp.where(et < 2, col * 2 + et, SDUMP)
    ss1 = jnp.where(et >= 2, col * 2 + (et - 2), SDUMP)
    ss = jnp.concatenate([ss0, ss1])
    colp = jnp.concatenate([col, jnp.full((16,), N, jnp.int32)])
    etp = jnp.concatenate([et, jnp.zeros((16,), jnp.int32)])
    distp = jnp.concatenate([dist, jnp.zeros((16,), F32)])
    zg = jnp.zeros((640, HALF), F32)
    zs = jnp.zeros((1256, 16), F32)

    dt = _sc_stats(colp, etp, distp, zs)
    st0 = dt[:N]
    st1 = dt[DROWS:DROWS + N]

    kr = jnp.kron(jnp.eye(4, dtype=F32), jnp.ones((1, RANK), F32))
    R1 = jnp.concatenate([kr, jnp.zeros((12, 64), F32)], axis=0)
    R2 = jnp.concatenate([jnp.zeros((4, 64), F32), kr,
                          jnp.zeros((8, 64), F32)], axis=0)

    bsT = [jnp.reshape(B_w[l][:, :, :HID], (T * RANK, HID)).T
           for l in range(2)]
    node, p4 = _tc_a(h, embed_w.T, embed_b.reshape(1, HID), bsT[0])

    for l in range(2):
        ncat = jnp.concatenate([node[:, :HALF], node[:, HALF:]], axis=0)
        p4r = p4.reshape(N * T, RANK)
        G2, S2 = _sc_layer(ncat, p4r, gg, col, sg, ss, zg, zs)
        G = jnp.concatenate([G2[:N], G2[GROWS:GROWS + N]], axis=1)
        S = jnp.concatenate(
            [S2[:2 * N].reshape(N, 2, RANK),
             S2[SROWS:SROWS + 2 * N].reshape(N, 2, RANK)],
            axis=1).reshape(N, T * RANK)

        WsT = Ws_w[l][:, :HID].T
        WtT = Ws_w[l][:, HID:2 * HID].T
        wd = Ws_w[l][:, 2 * HID].reshape(1, HID)
        BtT = jnp.reshape(B_w[l][:, :, HID:2 * HID], (T * RANK, HID)).T
        bd = B_w[l][:, :, 2 * HID].reshape(1, T * RANK)
        Acat = jnp.reshape(jnp.transpose(A_w[l], (0, 2, 1)), (T * RANK, HID))
        R3 = jnp.concatenate([A_b[l], jnp.zeros((12, HID), F32)], axis=0)

        araw, sums = _tc_b(G, node, S, st0, st1, WsT, WtT, wd, BtT, bd,
                           Acat, R1, R2, R3)
        bsnT = bsT[1] if l == 0 else jnp.zeros((HID, T * RANK), F32)
        node, p4 = _tc_c(araw, sums, node, bn_g[l].reshape(1, HID),
                         bn_b[l].reshape(1, HID), res_w[l].T,
                         upd_w[l][:, :HID].T, upd_w[l][:, HID:].T,
                         upd_b[l].reshape(1, HID), bsnT)
    return node


# speedup vs baseline: 6.4897x; 1.5371x over previous
import functools
import jax
import jax.numpy as jnp
from jax import lax
from jax.experimental import pallas as pl
from jax.experimental.pallas import tpu as pltpu
from jax.experimental.pallas import tpu_sc as plsc

N = 10000
E = 160000
HID = 256
T = 4
RANK = 16
HALF = 128
NS = 16
SROWS = 2 * N + 96
SDUMP = 2 * N
GROWS = 10240
DROWS = 10240
F32 = jnp.float32

_MESH = plsc.VectorSubcoreMesh(core_axis_name="c", subcore_axis_name="s")


def _stats_body(colp, etp, distp, zs, dt_out,
                dt_acc, cbuf, ebuf, dbuf, pay,
                cbuf_t, ebuf_t, dbuf_t, pay_t, sem):
    c = lax.axis_index("c")
    s = lax.axis_index("s")
    pltpu.sync_copy(zs.at[pl.ds(0, 640)], dt_acc.at[pl.ds(s * 640, 640)])
    plsc.subcore_barrier()
    base = (c * NS + s) * 5000
    ones16 = jnp.ones((16,), F32)
    iota16 = lax.iota(jnp.int32, 16)

    def chunk(j, carry):
        b = base + j * 128
        pltpu.sync_copy(colp.at[pl.ds(b, 128)], cbuf)
        pltpu.sync_copy(etp.at[pl.ds(b, 128)], ebuf)
        pltpu.sync_copy(distp.at[pl.ds(b, 128)], dbuf)
        pltpu.sync_copy(zs.at[pl.ds(0, 128)], pay)
        for g in range(8):
            kk = g * 16
            tv = ebuf[pl.ds(kk, 16)]
            dv = dbuf[pl.ds(kk, 16)]
            rv = iota16 + kk
            plsc.store_scatter(pay, [rv, tv], ones16)
            plsc.store_scatter(pay, [rv, tv + 4], dv)
        pltpu.sync_copy(pay, dt_acc.at[cbuf], add=True)
        return carry

    lax.fori_loop(0, 39, chunk, 0)
    b = base + 4992
    pltpu.sync_copy(colp.at[pl.ds(b, 16)], cbuf_t)
    pltpu.sync_copy(etp.at[pl.ds(b, 16)], ebuf_t)
    pltpu.sync_copy(distp.at[pl.ds(b, 16)], dbuf_t)
    pltpu.sync_copy(zs.at[pl.ds(0, 16)], pay_t)
    cv = jnp.where(iota16 < 8, cbuf_t[...], N)
    tv = ebuf_t[...]
    dv = dbuf_t[...]
    plsc.store_scatter(pay_t, [iota16, tv], ones16)
    plsc.store_scatter(pay_t, [iota16, tv + 4], dv)
    pltpu.sync_copy(pay_t, dt_acc.at[cv], add=True)
    plsc.subcore_barrier()
    pltpu.sync_copy(dt_acc.at[pl.ds(s * 640, 640)],
                    dt_out.at[pl.ds(c * DROWS + s * 640, 640)])


_sc_stats = functools.partial(
    pl.kernel, _stats_body,
    out_type=jax.ShapeDtypeStruct((2 * DROWS, 16), F32),
    mesh=_MESH,
    compiler_params=pltpu.CompilerParams(use_tc_tiling_on_sc=False, needs_layout_passes=False),
    scratch_types=[
        pltpu.VMEM_SHARED((DROWS, 16), F32),
        pltpu.VMEM((128,), jnp.int32),
        pltpu.VMEM((128,), jnp.int32),
        pltpu.VMEM((128,), F32),
        pltpu.VMEM((128, 16), F32),
        pltpu.VMEM((16,), jnp.int32),
        pltpu.VMEM((16,), jnp.int32),
        pltpu.VMEM((16,), F32),
        pltpu.VMEM((16, 16), F32),
        pltpu.SemaphoreType.DMA,
    ],
)()


def _layer_body(ncat, p4, gg, gs, sg, ss, zg, zs, g_out, s_out,
                g_acc, s_acc,
                gidx0, cidx0, sgidx0, ssidx0, gbuf0, pbuf0,
                gidx1, cidx1, sgidx1, ssidx1, gbuf1, pbuf1,
                gidx_t, cidx_t, sgidx_t, ssidx_t, gbuf_t, pbuf_t,
                semg0, semp0, semg1, semp1, semi1, semi2, semi3, semi4,
                semt):
    c = lax.axis_index("c")
    s = lax.axis_index("s")
    pltpu.sync_copy(zg, g_acc.at[pl.ds(s * 640, 640)])
    pltpu.sync_copy(zs, s_acc.at[pl.ds(s * 1256, 1256)])
    plsc.subcore_barrier()
    base = s * 10000
    cE = c * E
    gidx = (gidx0, gidx1)
    cidx = (cidx0, cidx1)
    sgidx = (sgidx0, sgidx1)
    ssidx = (ssidx0, ssidx1)
    gbuf = (gbuf0, gbuf1)
    pbuf = (pbuf0, pbuf1)
    semg = (semg0, semg1)
    semp = (semp0, semp1)

    def load_idx(j, p, sync):
        b = base + j * 64
        if sync:
            pltpu.sync_copy(gg.at[pl.ds(cE + b, 64)], gidx[p])
            pltpu.sync_copy(gs.at[pl.ds(b, 64)], cidx[p])
            pltpu.sync_copy(sg.at[pl.ds(b, 64)], sgidx[p])
            pltpu.sync_copy(ss.at[pl.ds(cE + b, 64)], ssidx[p])
        else:
            pltpu.async_copy(gg.at[pl.ds(cE + b, 64)], gidx[p], semi1)
            pltpu.async_copy(gs.at[pl.ds(b, 64)], cidx[p], semi2)
            pltpu.async_copy(sg.at[pl.ds(b, 64)], sgidx[p], semi3)
            pltpu.async_copy(ss.at[pl.ds(cE + b, 64)], ssidx[p], semi4)

    def wait_idx(p):
        pltpu.make_async_copy(gg.at[pl.ds(0, 64)], gidx[p], semi1).wait()
        pltpu.make_async_copy(gs.at[pl.ds(0, 64)], cidx[p], semi2).wait()
        pltpu.make_async_copy(sg.at[pl.ds(0, 64)], sgidx[p], semi3).wait()
        pltpu.make_async_copy(ss.at[pl.ds(0, 64)], ssidx[p], semi4).wait()

    def start_gather(p):
        pltpu.async_copy(ncat.at[gidx[p]], gbuf[p], semg[p])
        pltpu.async_copy(p4.at[sgidx[p]], pbuf[p], semp[p])

    def wait_gather(p):
        pltpu.make_async_copy(ncat.at[gidx[p]], gbuf[p], semg[p]).wait()
        pltpu.make_async_copy(p4.at[sgidx[p]], pbuf[p], semp[p]).wait()

    def scatter(p):
        pltpu.sync_copy(gbuf[p], g_acc.at[cidx[p]], add=True)
        pltpu.sync_copy(pbuf[p], s_acc.at[ssidx[p]], add=True)

    load_idx(0, 0, True)
    start_gather(0)
    load_idx(1, 1, False)

    def half(j, p, start_next, load_next):
        if start_next:
            wait_idx(1 - p)
            start_gather(1 - p)
        wait_gather(p)
        if load_next is not None:
            load_idx(load_next, p, False)
        scatter(p)

    def pair(i, carry):
        j = 2 * i
        half(j, 0, True, None)
        load_idx(j + 2, 0, False)
        half(j + 1, 1, True, None)
        load_idx(j + 3, 1, False)
        return carry

    lax.fori_loop(0, 77, pair, 0)
    wait_idx(1)
    start_gather(1)
    wait_gather(0)
    scatter(0)
    wait_gather(1)
    scatter(1)

    b = base + 9984
    pltpu.sync_copy(gg.at[pl.ds(cE + b, 16)], gidx_t)
    pltpu.sync_copy(gs.at[pl.ds(b, 16)], cidx_t)
    pltpu.sync_copy(sg.at[pl.ds(b, 16)], sgidx_t)
    pltpu.sync_copy(ss.at[pl.ds(cE + b, 16)], ssidx_t)
    pltpu.async_copy(ncat.at[gidx_t], gbuf_t, semt).wait()
    pltpu.sync_copy(gbuf_t, g_acc.at[cidx_t], add=True)
    pltpu.async_copy(p4.at[sgidx_t], pbuf_t, semt).wait()
    pltpu.sync_copy(pbuf_t, s_acc.at[ssidx_t], add=True)
    plsc.subcore_barrier()
    pltpu.sync_copy(g_acc.at[pl.ds(s * 640, 640)],
                    g_out.at[pl.ds(c * GROWS + s * 640, 640)])
    pltpu.sync_copy(s_acc.at[pl.ds(s * 1256, 1256)],
                    s_out.at[pl.ds(c * SROWS + s * 1256, 1256)])


_sc_layer = functools.partial(
    pl.kernel, _layer_body,
    out_type=(jax.ShapeDtypeStruct((2 * GROWS, HALF), F32),
              jax.ShapeDtypeStruct((2 * SROWS, 16), F32)),
    mesh=_MESH,
    compiler_params=pltpu.CompilerParams(use_tc_tiling_on_sc=False, needs_layout_passes=False),
    scratch_types=(
        [pltpu.VMEM_SHARED((GROWS, HALF), F32),
         pltpu.VMEM_SHARED((SROWS, 16), F32)]
        + 2 * [pltpu.VMEM((64,), jnp.int32), pltpu.VMEM((64,), jnp.int32),
               pltpu.VMEM((64,), jnp.int32), pltpu.VMEM((64,), jnp.int32),
               pltpu.VMEM((64, HALF), F32), pltpu.VMEM((64, 16), F32)]
        + [pltpu.VMEM((16,), jnp.int32), pltpu.VMEM((16,), jnp.int32),
           pltpu.VMEM((16,), jnp.int32), pltpu.VMEM((16,), jnp.int32),
           pltpu.VMEM((16, HALF), F32), pltpu.VMEM((16, 16), F32)]
        + 9 * [pltpu.SemaphoreType.DMA]
    ),
)()


_TN = 1000
_GRID = N // _TN


def _full(shape):
    return pl.BlockSpec(shape, lambda i: (0, 0))


def _rows(width):
    return pl.BlockSpec((_TN, width), lambda i: (i, 0))


def _tca_body(h, ewT, eb, bsT, node_o, p4_o):
    node = jnp.dot(h[...], ewT[...], preferred_element_type=F32) + eb[...]
    node_o[...] = node
    p4_o[...] = jnp.dot(node, bsT[...], preferred_element_type=F32)


def _tc_a(h, ewT, eb, bsT):
    return pl.pallas_call(
        _tca_body,
        grid=(_GRID,),
        in_specs=[_rows(HID), _full((HID, HID)), _full((1, HID)),
                  _full((HID, 64))],
        out_specs=[_rows(HID), _rows(64)],
        out_shape=[jax.ShapeDtypeStruct((N, HID), F32),
                   jax.ShapeDtypeStruct((N, 64), F32)],
    )(h, ewT, eb, bsT)


def _tcb_body(G, node, S, st0, st1, WsT, WtT, wd, BtT, bd, Acat, R1, R2, R3,
              araw_o, sums_o):
    i = pl.program_id(0)
    st = st0[...] + st1[...]
    cnt = jnp.sum(st[:, 0:4], axis=1, keepdims=True)
    dsum = jnp.sum(st[:, 4:8], axis=1, keepdims=True)
    denom = jnp.maximum(cnt, 1.0)
    nodev = node[...]
    ap = (jnp.dot(G[...], WsT[...], preferred_element_type=F32)
          + cnt * jnp.dot(nodev, WtT[...], preferred_element_type=F32)
          + dsum * wd[...])
    Q4 = jnp.dot(nodev, BtT[...], preferred_element_type=F32)
    Z = (S[...] + jnp.dot(st, R1[...], preferred_element_type=F32) * Q4
         + jnp.dot(st, R2[...], preferred_element_type=F32) * bd[...])
    lr = (jnp.dot(Z, Acat[...], preferred_element_type=F32)
          + jnp.dot(st, R3[...], preferred_element_type=F32))
    araw = (ap + lr) / denom
    araw_o[...] = araw
    s1 = jnp.sum(araw, axis=0, keepdims=True)
    s2 = jnp.sum(araw * araw, axis=0, keepdims=True)
    pad = jnp.concatenate([s1, s2, jnp.zeros((6, HID), F32)], axis=0)

    @pl.when(i == 0)
    def _():
        sums_o[...] = pad

    @pl.when(i > 0)
    def _():
        sums_o[...] = sums_o[...] + pad


def _tc_b(G, node, S, st0, st1, WsT, WtT, wd, BtT, bd, Acat, R1, R2, R3):
    return pl.pallas_call(
        _tcb_body,
        grid=(_GRID,),
        in_specs=[_rows(HID), _rows(HID), _rows(64), _rows(16), _rows(16),
                  _full((HID, HID)), _full((HID, HID)), _full((1, HID)),
                  _full((HID, 64)), _full((1, 64)), _full((64, HID)),
                  _full((16, 64)), _full((16, 64)), _full((16, HID))],
        out_specs=[_rows(HID), pl.BlockSpec((8, HID), lambda i: (0, 0))],
        out_shape=[jax.ShapeDtypeStruct((N, HID), F32),
                   jax.ShapeDtypeStruct((8, HID), F32)],
    )(G, node, S, st0, st1, WsT, WtT, wd, BtT, bd, Acat, R1, R2, R3)


def _tcc_body(araw, sums, node, bng, bnb, resT, unT, uaT, ub, bsnT,
              node_o, p4_o):
    mu = sums[0:1, :] * (1.0 / N)
    var = sums[1:2, :] * (1.0 / N) - mu * mu
    inv = lax.rsqrt(var + 1e-5)
    aggr = (araw[...] - mu) * (inv * bng[...]) + bnb[...]
    nodev = node[...]
    pre = (jnp.dot(nodev, unT[...], preferred_element_type=F32)
           + jnp.dot(aggr, uaT[...], preferred_element_type=F32) + ub[...])
    nn = (jnp.dot(nodev, resT[...], preferred_element_type=F32)
          + jnp.maximum(pre, 0.0))
    node_o[...] = nn
    p4_o[...] = jnp.dot(nn, bsnT[...], preferred_element_type=F32)


def _tc_c(araw, sums, node, bng, bnb, resT, unT, uaT, ub, bsnT):
    return pl.pallas_call(
        _tcc_body,
        grid=(_GRID,),
        in_specs=[_rows(HID), pl.BlockSpec((8, HID), lambda i: (0, 0)),
                  _rows(HID), _full((1, HID)), _full((1, HID)),
                  _full((HID, HID)), _full((HID, HID)), _full((HID, HID)),
                  _full((1, HID)), _full((HID, 64))],
        out_specs=[_rows(HID), _rows(64)],
        out_shape=[jax.ShapeDtypeStruct((N, HID), F32),
                   jax.ShapeDtypeStruct((N, 64), F32)],
    )(araw, sums, node, bng, bnb, resT, unT, uaT, ub, bsnT)


def kernel(x, h, edge_attr, embed_w, embed_b, A_w, A_b, B_w, Ws_w, res_w,
           upd_w, upd_b, bn_g, bn_b, edge_index, batch):
    row = edge_index[0].astype(jnp.int32)
    col = edge_index[1].astype(jnp.int32)
    et = edge_attr[:, 0].astype(jnp.int32)
    dist = edge_attr[:, 1].astype(F32)

    gg = jnp.concatenate([row, row + N])
    sg = row * T + et
    ss0 = jnp.where(et < 2, col * 2 + et, SDUMP)
    ss1 = jnp.where(et >= 2, col * 2 + (et - 2), SDUMP)
    ss = jnp.concatenate([ss0, ss1])
    colp = jnp.concatenate([col, jnp.full((16,), N, jnp.int32)])
    etp = jnp.concatenate([et, jnp.zeros((16,), jnp.int32)])
    distp = jnp.concatenate([dist, jnp.zeros((16,), F32)])
    zg = jnp.zeros((640, HALF), F32)
    zs = jnp.zeros((1256, 16), F32)

    dt = _sc_stats(colp, etp, distp, zs)
    st0 = dt[:N]
    st1 = dt[DROWS:DROWS + N]

    kr = jnp.kron(jnp.eye(4, dtype=F32), jnp.ones((1, RANK), F32))
    R1 = jnp.concatenate([kr, jnp.zeros((12, 64), F32)], axis=0)
    R2 = jnp.concatenate([jnp.zeros((4, 64), F32), kr,
                          jnp.zeros((8, 64), F32)], axis=0)

    bsT = [jnp.reshape(B_w[l][:, :, :HID], (T * RANK, HID)).T
           for l in range(2)]
    node, p4 = _tc_a(h, embed_w.T, embed_b.reshape(1, HID), bsT[0])

    for l in range(2):
        ncat = jnp.concatenate([node[:, :HALF], node[:, HALF:]], axis=0)
        p4r = p4.reshape(N * T, RANK)
        G2, S2 = _sc_layer(ncat, p4r, gg, col, sg, ss, zg, zs)
        G = jnp.concatenate([G2[:N], G2[GROWS:GROWS + N]], axis=1)
        S = jnp.concatenate(
            [S2[:2 * N].reshape(N, 2, RANK),
             S2[SROWS:SROWS + 2 * N].reshape(N, 2, RANK)],
            axis=1).reshape(N, T * RANK)

        WsT = Ws_w[l][:, :HID].T
        WtT = Ws_w[l][:, HID:2 * HID].T
        wd = Ws_w[l][:, 2 * HID].reshape(1, HID)
        BtT = jnp.reshape(B_w[l][:, :, HID:2 * HID], (T * RANK, HID)).T
        bd = B_w[l][:, :, 2 * HID].reshape(1, T * RANK)
        Acat = jnp.reshape(jnp.transpose(A_w[l], (0, 2, 1)), (T * RANK, HID))
        R3 = jnp.concatenate([A_b[l], jnp.zeros((12, HID), F32)], axis=0)

        araw, sums = _tc_b(G, node, S, st0, st1, WsT, WtT, wd, BtT, bd,
                           Acat, R1, R2, R3)
        bsnT = bsT[1] if l == 0 else jnp.zeros((HID, T * RANK), F32)
        node, p4 = _tc_c(araw, sums, node, bn_g[l].reshape(1, HID),
                         bn_b[l].reshape(1, HID), res_w[l].T,
                         upd_w[l][:, :HID].T, upd_w[l][:, HID:].T,
                         upd_b[l].reshape(1, HID), bsnT)
    return node


# speedup vs baseline: 6.8653x; 1.0579x over previous
import functools
import jax
import jax.numpy as jnp
from jax import lax
from jax.experimental import pallas as pl
from jax.experimental.pallas import tpu as pltpu
from jax.experimental.pallas import tpu_sc as plsc

N = 10000
E = 160000
HID = 256
T = 4
RANK = 16
HALF = 128
NS = 16
SROWS = 2 * N + 96
SDUMP = 2 * N
GROWS = 10240
DROWS = 10240
F32 = jnp.float32

_MESH = plsc.VectorSubcoreMesh(core_axis_name="c", subcore_axis_name="s")


def _stats_body(colp, etp, distp, zs, dt_out,
                dt_acc, cbuf, ebuf, dbuf, pay,
                cbuf_t, ebuf_t, dbuf_t, pay_t, sem):
    c = lax.axis_index("c")
    s = lax.axis_index("s")
    pltpu.sync_copy(zs.at[pl.ds(0, 640)], dt_acc.at[pl.ds(s * 640, 640)])
    plsc.subcore_barrier()
    base = (c * NS + s) * 5000
    ones16 = jnp.ones((16,), F32)
    iota16 = lax.iota(jnp.int32, 16)

    def chunk(j, carry):
        b = base + j * 128
        pltpu.sync_copy(colp.at[pl.ds(b, 128)], cbuf)
        pltpu.sync_copy(etp.at[pl.ds(b, 128)], ebuf)
        pltpu.sync_copy(distp.at[pl.ds(b, 128)], dbuf)
        pltpu.sync_copy(zs.at[pl.ds(0, 128)], pay)
        for g in range(8):
            kk = g * 16
            tv = ebuf[pl.ds(kk, 16)]
            dv = dbuf[pl.ds(kk, 16)]
            rv = iota16 + kk
            plsc.store_scatter(pay, [rv, tv], ones16)
            plsc.store_scatter(pay, [rv, tv + 4], dv)
        pltpu.sync_copy(pay, dt_acc.at[cbuf], add=True)
        return carry

    lax.fori_loop(0, 39, chunk, 0)
    b = base + 4992
    pltpu.sync_copy(colp.at[pl.ds(b, 16)], cbuf_t)
    pltpu.sync_copy(etp.at[pl.ds(b, 16)], ebuf_t)
    pltpu.sync_copy(distp.at[pl.ds(b, 16)], dbuf_t)
    pltpu.sync_copy(zs.at[pl.ds(0, 16)], pay_t)
    cv = jnp.where(iota16 < 8, cbuf_t[...], N)
    tv = ebuf_t[...]
    dv = dbuf_t[...]
    plsc.store_scatter(pay_t, [iota16, tv], ones16)
    plsc.store_scatter(pay_t, [iota16, tv + 4], dv)
    pltpu.sync_copy(pay_t, dt_acc.at[cv], add=True)
    plsc.subcore_barrier()
    pltpu.sync_copy(dt_acc.at[pl.ds(s * 640, 640)],
                    dt_out.at[pl.ds(c * DROWS + s * 640, 640)])


_sc_stats = functools.partial(
    pl.kernel, _stats_body,
    out_type=jax.ShapeDtypeStruct((2 * DROWS, 16), F32),
    mesh=_MESH,
    compiler_params=pltpu.CompilerParams(use_tc_tiling_on_sc=False, needs_layout_passes=False),
    scratch_types=[
        pltpu.VMEM_SHARED((DROWS, 16), F32),
        pltpu.VMEM((128,), jnp.int32),
        pltpu.VMEM((128,), jnp.int32),
        pltpu.VMEM((128,), F32),
        pltpu.VMEM((128, 16), F32),
        pltpu.VMEM((16,), jnp.int32),
        pltpu.VMEM((16,), jnp.int32),
        pltpu.VMEM((16,), F32),
        pltpu.VMEM((16, 16), F32),
        pltpu.SemaphoreType.DMA,
    ],
)()


def _layer_body(ncat, p4, gr, gs, sg, ss, zg, zs, g_out, s_out,
                g_acc, s_acc,
                gidx0, cidx0, sgidx0, ssidx0, gbuf0, pbuf0,
                gidx1, cidx1, sgidx1, ssidx1, gbuf1, pbuf1,
                gidx_t, cidx_t, sgidx_t, ssidx_t, gbuf_t, pbuf_t,
                semg0, semp0, semg1, semp1, semi1, semi2, semi3, semi4,
                semt):
    c = lax.axis_index("c")
    s = lax.axis_index("s")
    pltpu.sync_copy(zg, g_acc.at[pl.ds(s * 640, 640)])
    pltpu.sync_copy(zs, s_acc.at[pl.ds(s * 1256, 1256)])
    plsc.subcore_barrier()
    base = s * 10000
    cE = c * E
    gview = g_out.at[:, pl.ds(c * HALF, HALF)]
    gidx = (gidx0, gidx1)
    cidx = (cidx0, cidx1)
    sgidx = (sgidx0, sgidx1)
    ssidx = (ssidx0, ssidx1)
    gbuf = (gbuf0, gbuf1)
    pbuf = (pbuf0, pbuf1)
    semg = (semg0, semg1)
    semp = (semp0, semp1)

    def load_idx(j, p, sync):
        b = base + j * 64
        if sync:
            pltpu.sync_copy(gr.at[pl.ds(cE + b, 64)], gidx[p])
            pltpu.sync_copy(gs.at[pl.ds(b, 64)], cidx[p])
            pltpu.sync_copy(sg.at[pl.ds(b, 64)], sgidx[p])
            pltpu.sync_copy(ss.at[pl.ds(cE + b, 64)], ssidx[p])
        else:
            pltpu.async_copy(gr.at[pl.ds(cE + b, 64)], gidx[p], semi1)
            pltpu.async_copy(gs.at[pl.ds(b, 64)], cidx[p], semi2)
            pltpu.async_copy(sg.at[pl.ds(b, 64)], sgidx[p], semi3)
            pltpu.async_copy(ss.at[pl.ds(cE + b, 64)], ssidx[p], semi4)

    def wait_idx(p):
        pltpu.make_async_copy(gr.at[pl.ds(0, 64)], gidx[p], semi1).wait()
        pltpu.make_async_copy(gs.at[pl.ds(0, 64)], cidx[p], semi2).wait()
        pltpu.make_async_copy(sg.at[pl.ds(0, 64)], sgidx[p], semi3).wait()
        pltpu.make_async_copy(ss.at[pl.ds(0, 64)], ssidx[p], semi4).wait()

    def start_gather(p):
        pltpu.async_copy(ncat.at[gidx[p]], gbuf[p], semg[p])
        pltpu.async_copy(p4.at[sgidx[p]], pbuf[p], semp[p])

    def wait_gather(p):
        pltpu.make_async_copy(ncat.at[gidx[p]], gbuf[p], semg[p]).wait()
        pltpu.make_async_copy(p4.at[sgidx[p]], pbuf[p], semp[p]).wait()

    def scatter(p):
        pltpu.sync_copy(gbuf[p], g_acc.at[cidx[p]], add=True)
        pltpu.sync_copy(pbuf[p], s_acc.at[ssidx[p]], add=True)

    load_idx(0, 0, True)
    start_gather(0)
    load_idx(1, 1, False)

    def half(j, p):
        wait_idx(1 - p)
        start_gather(1 - p)
        wait_gather(p)
        load_idx(j + 2, p, False)
        scatter(p)

    def pair(i, carry):
        j = 2 * i
        half(j, 0)
        half(j + 1, 1)
        return carry

    lax.fori_loop(0, 77, pair, 0)
    wait_idx(1)
    start_gather(1)
    wait_gather(0)
    scatter(0)
    wait_gather(1)
    scatter(1)

    b = base + 9984
    pltpu.sync_copy(gr.at[pl.ds(cE + b, 16)], gidx_t)
    pltpu.sync_copy(gs.at[pl.ds(b, 16)], cidx_t)
    pltpu.sync_copy(sg.at[pl.ds(b, 16)], sgidx_t)
    pltpu.sync_copy(ss.at[pl.ds(cE + b, 16)], ssidx_t)
    pltpu.async_copy(ncat.at[gidx_t], gbuf_t, semt).wait()
    pltpu.sync_copy(gbuf_t, g_acc.at[cidx_t], add=True)
    pltpu.async_copy(p4.at[sgidx_t], pbuf_t, semt).wait()
    pltpu.sync_copy(pbuf_t, s_acc.at[ssidx_t], add=True)
    plsc.subcore_barrier()

    @pl.when(s < 15)
    def _():
        pltpu.sync_copy(g_acc.at[pl.ds(s * 632, 632)],
                        gview.at[pl.ds(s * 632, 632)])

    @pl.when(s == 15)
    def _():
        pltpu.sync_copy(g_acc.at[pl.ds(9480, 520)],
                        gview.at[pl.ds(9480, 520)])

    pltpu.sync_copy(s_acc.at[pl.ds(s * 1256, 1256)],
                    s_out.at[pl.ds(c * SROWS + s * 1256, 1256)])


_sc_layer = functools.partial(
    pl.kernel, _layer_body,
    out_type=(jax.ShapeDtypeStruct((N, HID), F32),
              jax.ShapeDtypeStruct((2 * SROWS, 16), F32)),
    mesh=_MESH,
    compiler_params=pltpu.CompilerParams(use_tc_tiling_on_sc=False, needs_layout_passes=False),
    scratch_types=(
        [pltpu.VMEM_SHARED((GROWS, HALF), F32),
         pltpu.VMEM_SHARED((SROWS, 16), F32)]
        + 2 * [pltpu.VMEM((64,), jnp.int32), pltpu.VMEM((64,), jnp.int32),
               pltpu.VMEM((64,), jnp.int32), pltpu.VMEM((64,), jnp.int32),
               pltpu.VMEM((64, HALF), F32), pltpu.VMEM((64, 16), F32)]
        + [pltpu.VMEM((16,), jnp.int32), pltpu.VMEM((16,), jnp.int32),
           pltpu.VMEM((16,), jnp.int32), pltpu.VMEM((16,), jnp.int32),
           pltpu.VMEM((16, HALF), F32), pltpu.VMEM((16, 16), F32)]
        + 9 * [pltpu.SemaphoreType.DMA]
    ),
)()


_TN = 1000
_GRID = N // _TN


def _full(shape):
    return pl.BlockSpec(shape, lambda i: (0, 0))


def _rows(width):
    return pl.BlockSpec((_TN, width), lambda i: (i, 0))


def _tca_body(h, ewT, eb, bsT, node_o, p4_o):
    node = jnp.dot(h[...], ewT[...], preferred_element_type=F32) + eb[...]
    node_o[...] = node
    p4_o[...] = jnp.dot(node, bsT[...], preferred_element_type=F32)


def _tc_a(h, ewT, eb, bsT):
    return pl.pallas_call(
        _tca_body,
        grid=(_GRID,),
        in_specs=[_rows(HID), _full((HID, HID)), _full((1, HID)),
                  _full((HID, 64))],
        out_specs=[_rows(HID), _rows(64)],
        out_shape=[jax.ShapeDtypeStruct((N, HID), F32),
                   jax.ShapeDtypeStruct((N, 64), F32)],
    )(h, ewT, eb, bsT)


def _tcb_body(G, node, S, st0, st1, WsT, WtT, wd, BtT, bd, Acat, R1, R2, R3,
              araw_o, sums_o):
    i = pl.program_id(0)
    st = st0[...] + st1[...]
    cnt = jnp.sum(st[:, 0:4], axis=1, keepdims=True)
    dsum = jnp.sum(st[:, 4:8], axis=1, keepdims=True)
    denom = jnp.maximum(cnt, 1.0)
    nodev = node[...]
    ap = (jnp.dot(G[...], WsT[...], preferred_element_type=F32)
          + cnt * jnp.dot(nodev, WtT[...], preferred_element_type=F32)
          + dsum * wd[...])
    Q4 = jnp.dot(nodev, BtT[...], preferred_element_type=F32)
    Z = (S[...] + jnp.dot(st, R1[...], preferred_element_type=F32) * Q4
         + jnp.dot(st, R2[...], preferred_element_type=F32) * bd[...])
    lr = (jnp.dot(Z, Acat[...], preferred_element_type=F32)
          + jnp.dot(st, R3[...], preferred_element_type=F32))
    araw = (ap + lr) / denom
    araw_o[...] = araw
    s1 = jnp.sum(araw, axis=0, keepdims=True)
    s2 = jnp.sum(araw * araw, axis=0, keepdims=True)
    pad = jnp.concatenate([s1, s2, jnp.zeros((6, HID), F32)], axis=0)

    @pl.when(i == 0)
    def _():
        sums_o[...] = pad

    @pl.when(i > 0)
    def _():
        sums_o[...] = sums_o[...] + pad


def _tc_b(G, node, S, st0, st1, WsT, WtT, wd, BtT, bd, Acat, R1, R2, R3):
    return pl.pallas_call(
        _tcb_body,
        grid=(_GRID,),
        in_specs=[_rows(HID), _rows(HID), _rows(64), _rows(16), _rows(16),
                  _full((HID, HID)), _full((HID, HID)), _full((1, HID)),
                  _full((HID, 64)), _full((1, 64)), _full((64, HID)),
                  _full((16, 64)), _full((16, 64)), _full((16, HID))],
        out_specs=[_rows(HID), pl.BlockSpec((8, HID), lambda i: (0, 0))],
        out_shape=[jax.ShapeDtypeStruct((N, HID), F32),
                   jax.ShapeDtypeStruct((8, HID), F32)],
    )(G, node, S, st0, st1, WsT, WtT, wd, BtT, bd, Acat, R1, R2, R3)


def _tcc_body(araw, sums, node, bng, bnb, resT, unT, uaT, ub, bsnT,
              node_o, p4_o):
    mu = sums[0:1, :] * (1.0 / N)
    var = sums[1:2, :] * (1.0 / N) - mu * mu
    inv = lax.rsqrt(var + 1e-5)
    aggr = (araw[...] - mu) * (inv * bng[...]) + bnb[...]
    nodev = node[...]
    pre = (jnp.dot(nodev, unT[...], preferred_element_type=F32)
           + jnp.dot(aggr, uaT[...], preferred_element_type=F32) + ub[...])
    nn = (jnp.dot(nodev, resT[...], preferred_element_type=F32)
          + jnp.maximum(pre, 0.0))
    node_o[...] = nn
    p4_o[...] = jnp.dot(nn, bsnT[...], preferred_element_type=F32)


def _tc_c(araw, sums, node, bng, bnb, resT, unT, uaT, ub, bsnT):
    return pl.pallas_call(
        _tcc_body,
        grid=(_GRID,),
        in_specs=[_rows(HID), pl.BlockSpec((8, HID), lambda i: (0, 0)),
                  _rows(HID), _full((1, HID)), _full((1, HID)),
                  _full((HID, HID)), _full((HID, HID)), _full((HID, HID)),
                  _full((1, HID)), _full((HID, 64))],
        out_specs=[_rows(HID), _rows(64)],
        out_shape=[jax.ShapeDtypeStruct((N, HID), F32),
                   jax.ShapeDtypeStruct((N, 64), F32)],
    )(araw, sums, node, bng, bnb, resT, unT, uaT, ub, bsnT)


def kernel(x, h, edge_attr, embed_w, embed_b, A_w, A_b, B_w, Ws_w, res_w,
           upd_w, upd_b, bn_g, bn_b, edge_index, batch):
    row = edge_index[0].astype(jnp.int32)
    col = edge_index[1].astype(jnp.int32)
    et = edge_attr[:, 0].astype(jnp.int32)
    dist = edge_attr[:, 1].astype(F32)

    gg = jnp.concatenate([row, row + N])
    sg = row * T + et
    ss0 = jnp.where(et < 2, col * 2 + et, SDUMP)
    ss1 = jnp.where(et >= 2, col * 2 + (et - 2), SDUMP)
    ss = jnp.concatenate([ss0, ss1])
    colp = jnp.concatenate([col, jnp.full((16,), N, jnp.int32)])
    etp = jnp.concatenate([et, jnp.zeros((16,), jnp.int32)])
    distp = jnp.concatenate([dist, jnp.zeros((16,), F32)])
    zg = jnp.zeros((640, HALF), F32)
    zs = jnp.zeros((1256, 16), F32)

    dt = _sc_stats(colp, etp, distp, zs)
    st0 = dt[:N]
    st1 = dt[DROWS:DROWS + N]

    kr = jnp.kron(jnp.eye(4, dtype=F32), jnp.ones((1, RANK), F32))
    R1 = jnp.concatenate([kr, jnp.zeros((12, 64), F32)], axis=0)
    R2 = jnp.concatenate([jnp.zeros((4, 64), F32), kr,
                          jnp.zeros((8, 64), F32)], axis=0)

    bsT = [jnp.reshape(B_w[l][:, :, :HID], (T * RANK, HID)).T
           for l in range(2)]
    node, p4 = _tc_a(h, embed_w.T, embed_b.reshape(1, HID), bsT[0])

    for l in range(2):
        ncat = jnp.concatenate([node[:, :HALF], node[:, HALF:]], axis=0)
        p4r = p4.reshape(N * T, RANK)
        G, S2 = _sc_layer(ncat, p4r, gg, col, sg, ss, zg, zs)
        S = jnp.concatenate(
            [S2[:2 * N].reshape(N, 2, RANK),
             S2[SROWS:SROWS + 2 * N].reshape(N, 2, RANK)],
            axis=1).reshape(N, T * RANK)

        WsT = Ws_w[l][:, :HID].T
        WtT = Ws_w[l][:, HID:2 * HID].T
        wd = Ws_w[l][:, 2 * HID].reshape(1, HID)
        BtT = jnp.reshape(B_w[l][:, :, HID:2 * HID], (T * RANK, HID)).T
        bd = B_w[l][:, :, 2 * HID].reshape(1, T * RANK)
        Acat = jnp.reshape(jnp.transpose(A_w[l], (0, 2, 1)), (T * RANK, HID))
        R3 = jnp.concatenate([A_b[l], jnp.zeros((12, HID), F32)], axis=0)

        araw, sums = _tc_b(G, node, S, st0, st1, WsT, WtT, wd, BtT, bd,
                           Acat, R1, R2, R3)
        bsnT = bsT[1] if l == 0 else jnp.zeros((HID, T * RANK), F32)
        node, p4 = _tc_c(araw, sums, node, bn_g[l].reshape(1, HID),
                         bn_b[l].reshape(1, HID), res_w[l].T,
                         upd_w[l][:, :HID].T, upd_w[l][:, HID:].T,
                         upd_b[l].reshape(1, HID), bsnT)
    return node
